# baseline (speedup 1.0000x reference)
import jax
import jax.numpy as jnp
from jax.experimental import pallas as pl
from jax.experimental.pallas import tpu as pltpu

_NUM_NODES = 1000000
_EMBED_DIM = 64
_WIDE = 512
_WROWS = _NUM_NODES * _EMBED_DIM // _WIDE
_BLOCK_ROWS = 5000
_GRID = _WROWS // _BLOCK_ROWS


def _copy_body(w_ref, o_ref):
    o_ref[...] = w_ref[...]


def kernel(weight):
    wide = weight.reshape(_WROWS, _WIDE)
    out = pl.pallas_call(
        _copy_body,
        out_shape=jax.ShapeDtypeStruct((_WROWS, _WIDE), jnp.float32),
        grid=(_GRID,),
        in_specs=[pl.BlockSpec((_BLOCK_ROWS, _WIDE), lambda i: (i, 0))],
        out_specs=pl.BlockSpec((_BLOCK_ROWS, _WIDE), lambda i: (i, 0)),
    )(wide)
    return out.reshape(_NUM_NODES, _EMBED_DIM)
